# baseline (speedup 1.0000x reference)
import jax
import jax.numpy as jnp
from jax.experimental import pallas as pl
from jax.experimental.pallas import tpu as pltpu

MAP_H, MAP_W = 128, 64
BK = 1024


def _som_kernel(x_ref, w_ref, out_ref, s_ref, m_ref, d_ref):
    p = pl.program_id(0)
    k = pl.program_id(1)

    @pl.when(p == 0)
    def _scores():
        x = x_ref[...]
        w = w_ref[...]
        xw = jax.lax.dot_general(
            x, w, (((1,), (1,)), ((), ())), preferred_element_type=jnp.float32
        )
        w2 = w * w
        ones_d = jnp.ones((1, w.shape[1]), dtype=jnp.float32)
        wn2 = jax.lax.dot_general(
            ones_d, w2, (((1,), (1,)), ((), ())),
            preferred_element_type=jnp.float32,
        )
        xn2 = jnp.sum(x * x, axis=1, keepdims=True)
        d2 = jnp.maximum(wn2 + xn2 - 2.0 * xw, 0.0)
        s = -10.0 * jnp.sqrt(d2)
        s_ref[:, pl.ds(k * BK, BK)] = s
        m_blk = jnp.max(s, axis=1, keepdims=True)
        e_blk = jnp.sum(jnp.exp(s - m_blk), axis=1, keepdims=True)

        @pl.when(k == 0)
        def _init():
            m_ref[...] = jnp.broadcast_to(m_blk, m_ref.shape)
            d_ref[...] = jnp.broadcast_to(e_blk, d_ref.shape)

        @pl.when(k > 0)
        def _update():
            m_old = m_ref[...]
            m_new = jnp.maximum(m_old, m_blk)
            d_ref[...] = (d_ref[...] * jnp.exp(m_old - m_new)
                          + e_blk * jnp.exp(m_blk - m_new))
            m_ref[...] = m_new

    @pl.when(p == 1)
    def _normalize():
        m = m_ref[:, 0:1]
        inv = 1.0 / d_ref[:, 0:1]
        s = s_ref[:, pl.ds(k * BK, BK)]
        out_ref[...] = jnp.exp(s - m) * inv


def kernel(input_signal, weight_matrix):
    b, d = input_signal.shape
    kk = weight_matrix.shape[0]
    nk = kk // BK
    out = pl.pallas_call(
        _som_kernel,
        grid=(2, nk),
        in_specs=[
            pl.BlockSpec((b, d), lambda p, k: (0, 0)),
            pl.BlockSpec((BK, d), lambda p, k: (jnp.where(p == 0, k, nk - 1), 0)),
        ],
        out_specs=pl.BlockSpec((b, BK), lambda p, k: (0, jnp.where(p == 0, 0, k))),
        out_shape=jax.ShapeDtypeStruct((b, kk), jnp.float32),
        scratch_shapes=[
            pltpu.VMEM((b, kk), jnp.float32),
            pltpu.VMEM((b, 128), jnp.float32),
            pltpu.VMEM((b, 128), jnp.float32),
        ],
    )(input_signal, weight_matrix)
    return out.reshape(b, MAP_H, MAP_W)


# speedup vs baseline: 1.3888x; 1.3888x over previous
import jax
import jax.numpy as jnp
from jax.experimental import pallas as pl

MAP_H, MAP_W = 128, 64


def _som_kernel(x_ref, w_ref, out_ref):
    x = x_ref[...]
    w = w_ref[...]
    xw = jax.lax.dot_general(
        x, w, (((1,), (1,)), ((), ())), preferred_element_type=jnp.float32
    )
    w2 = w * w
    ones_d = jnp.ones((1, w.shape[1]), dtype=jnp.float32)
    wn2 = jax.lax.dot_general(
        ones_d, w2, (((1,), (1,)), ((), ())), preferred_element_type=jnp.float32
    )
    xn2 = jnp.sum(x * x, axis=1, keepdims=True)
    d2 = jnp.maximum(wn2 + xn2 - 2.0 * xw, 0.0)
    s = -10.0 * jnp.sqrt(d2)
    m = jnp.max(s, axis=1, keepdims=True)
    e = jnp.exp(s - m)
    out_ref[...] = e / jnp.sum(e, axis=1, keepdims=True)


def kernel(input_signal, weight_matrix):
    b = input_signal.shape[0]
    k = weight_matrix.shape[0]
    out = pl.pallas_call(
        _som_kernel,
        out_shape=jax.ShapeDtypeStruct((b, k), jnp.float32),
    )(input_signal, weight_matrix)
    return out.reshape(b, MAP_H, MAP_W)
